# baseline (speedup 1.0000x reference)
import functools

import jax
import jax.numpy as jnp
from jax import lax
from jax.experimental import pallas as pl
from jax.experimental.pallas import tpu as pltpu
from jax.experimental.pallas import tpu_sc as plsc

N_EMB = 3072
EMB_DIM = 8
D = 128
N_ROWS = 204800

NC, NS, L = 2, 16, 16
NW = NC * NS
ROWS_PER_W = N_ROWS // NW
SUB = 128
CHUNK = 256
N_CHUNKS = ROWS_PER_W // CHUNK
N_SUB = CHUNK // SUB


def _project_body(emb_ref, w_ref, b_ref, p_ref):
    w8 = w_ref[:, :EMB_DIM]
    p_ref[...] = lax.dot_general(
        emb_ref[...], w8,
        dimension_numbers=(((1,), (1,)), ((), ())),
        preferred_element_type=jnp.float32,
    ) + b_ref[...]


def _project(emb_table, W, b2d):
    return pl.pallas_call(
        _project_body,
        out_shape=jax.ShapeDtypeStruct((N_EMB, D), jnp.float32),
    )(emb_table, W, b2d)


_BCAST_DN = lax.GatherDimensionNumbers(
    offset_dims=(), collapsed_slice_dims=(0,), start_index_map=(0,))


def _lane_bcast(vec, lane):
    idx = jnp.full((L, 1), lane, dtype=jnp.int32)
    return lax.gather(vec, idx, _BCAST_DN, (1,),
                      mode=lax.GatherScatterMode.PROMISE_IN_BOUNDS)


def _sc_body(p_hbm, atom_hbm, t_hbm, m_hbm, wt_hbm, wm_hbm, out_hbm,
             p_sh, idx_all, t_all, m_all, rows0_v, rows1_v, wt_v, wm_v,
             sem_g0, sem_g1, sem_o0, sem_o1):
    sid = lax.axis_index("s")
    wid = sid * NC + lax.axis_index("c")
    base0 = wid * ROWS_PER_W
    rows = (rows0_v, rows1_v)
    sem_g = (sem_g0, sem_g1)
    sem_o = (sem_o0, sem_o1)

    @pl.when(sid == 0)
    def _():
        pltpu.sync_copy(p_hbm, p_sh)

    pltpu.sync_copy(wt_hbm, wt_v)
    pltpu.sync_copy(wm_hbm, wm_v)
    wts = [wt_v[pl.ds(j * L, L)] for j in range(D // L)]
    wms = [wm_v[pl.ds(j * L, L)] for j in range(D // L)]

    pltpu.sync_copy(atom_hbm.at[pl.ds(base0, ROWS_PER_W)], idx_all)
    pltpu.sync_copy(t_hbm.at[pl.ds(base0, ROWS_PER_W)], t_all)
    pltpu.sync_copy(m_hbm.at[pl.ds(base0, ROWS_PER_W)], m_all)
    plsc.subcore_barrier()

    def gather_add(ci, b):
        for s in range(N_SUB):
            pltpu.async_copy(
                p_sh.at[idx_all.at[pl.ds(ci * CHUNK + s * SUB, SUB)]],
                rows[b].at[pl.ds(s * SUB, SUB)],
                sem_g[b], add=True)

    def store_out(ci, b):
        pltpu.async_copy(
            rows[b], out_hbm.at[pl.ds(base0 + ci * CHUNK, CHUNK)], sem_o[b])

    def wait_gather(b):
        for s in range(N_SUB):
            pltpu.make_async_copy(
                p_sh.at[idx_all.at[pl.ds(0, SUB)]],
                rows[b].at[pl.ds(s * SUB, SUB)], sem_g[b]).wait()

    def wait_store(b):
        pltpu.make_async_copy(
            rows[b], out_hbm.at[pl.ds(0, CHUNK)], sem_o[b]).wait()

    def addend(ci, b):
        def group_body(g, c2):
            off = ci * CHUNK + g * L
            tv = t_all[pl.ds(off, L)]
            mv = m_all[pl.ds(off, L)]
            for r in range(L):
                t16 = _lane_bcast(tv, r)
                m16 = _lane_bcast(mv, r)
                row = g * L + r
                for j in range(D // L):
                    rows[b][row, pl.ds(j * L, L)] = (t16 * wts[j]
                                                     + m16 * wms[j])
            return c2

        lax.fori_loop(0, CHUNK // L, group_body, 0)

    @pl.loop(0, N_CHUNKS - 1, step=2)
    def chunk_pair(i0):
        @pl.when(i0 >= 2)
        def _():
            wait_store(0)
        addend(i0, 0)
        gather_add(i0, 0)

        @pl.when(i0 >= 1)
        def _():
            wait_gather(1)
            store_out(i0 - 1, 1)

        @pl.when(i0 >= 1)
        def _():
            wait_store(1)
        addend(i0 + 1, 1)
        gather_add(i0 + 1, 1)
        wait_gather(0)
        store_out(i0, 0)

    wait_store(0)
    addend(N_CHUNKS - 1, 0)
    gather_add(N_CHUNKS - 1, 0)
    wait_gather(1)
    store_out(N_CHUNKS - 2, 1)
    wait_gather(0)
    store_out(N_CHUNKS - 1, 0)
    wait_store(0)
    wait_store(1)


_sc_lookup = functools.partial(
    pl.kernel,
    out_type=jax.ShapeDtypeStruct((N_ROWS, D), jnp.float32),
    mesh=plsc.VectorSubcoreMesh(core_axis_name="c", subcore_axis_name="s"),
    scratch_types=[
        pltpu.VMEM_SHARED((N_EMB, D), jnp.float32),
        pltpu.VMEM((ROWS_PER_W,), jnp.int32),
        pltpu.VMEM((ROWS_PER_W,), jnp.float32),
        pltpu.VMEM((ROWS_PER_W,), jnp.float32),
        pltpu.VMEM((CHUNK, D), jnp.float32),
        pltpu.VMEM((CHUNK, D), jnp.float32),
        pltpu.VMEM((D,), jnp.float32),
        pltpu.VMEM((D,), jnp.float32),
        pltpu.SemaphoreType.DMA,
        pltpu.SemaphoreType.DMA,
        pltpu.SemaphoreType.DMA,
        pltpu.SemaphoreType.DMA,
    ],
)(_sc_body)


def kernel(atom, time, mag, emb_table, W, b):
    p = _project(emb_table, W, b.reshape(1, D))
    out = _sc_lookup(p, atom.reshape(-1), time.reshape(-1), mag.reshape(-1),
                     W[:, EMB_DIM], W[:, EMB_DIM + 1])
    return out


# speedup vs baseline: 1.2786x; 1.2786x over previous
import functools

import jax
import jax.numpy as jnp
import numpy as np
from jax import lax
from jax.experimental import pallas as pl
from jax.experimental.pallas import tpu as pltpu
from jax.experimental.pallas import tpu_sc as plsc

N_EMB = 3072
EMB_DIM = 8
D = 128
N_ROWS = 204800

NC, NS, L = 2, 16, 16
NW = NC * NS
ROWS_PER_W = N_ROWS // NW
SUB = 128
CHUNK = 128
N_CHUNKS = ROWS_PER_W // CHUNK
N_SUB = CHUNK // SUB

_PERM = np.empty((D,), dtype=np.int32)
for _kb in range(D // 32):
    for _j in range(16):
        _PERM[_kb * 32 + 2 * _j] = _kb * 32 + _j
        _PERM[_kb * 32 + 2 * _j + 1] = _kb * 32 + 16 + _j


def _project_body(emb_ref, w_ref, b_ref, p_ref):
    w8 = w_ref[:, :EMB_DIM]
    p_ref[...] = lax.dot_general(
        emb_ref[...], w8,
        dimension_numbers=(((1,), (1,)), ((), ())),
        preferred_element_type=jnp.float32,
    ) + b_ref[...]


def _project(emb_table, Wp, bp2d):
    return pl.pallas_call(
        _project_body,
        out_shape=jax.ShapeDtypeStruct((N_EMB, D), jnp.float32),
    )(emb_table, Wp, bp2d)


_BCAST_DN = lax.GatherDimensionNumbers(
    offset_dims=(), collapsed_slice_dims=(0,), start_index_map=(0,))


def _lane_bcast(vec, lane):
    idx = jnp.full((L, 1), lane, dtype=jnp.int32)
    return lax.gather(vec, idx, _BCAST_DN, (1,),
                      mode=lax.GatherScatterMode.PROMISE_IN_BOUNDS)


def _sc_body(p_hbm, atom_hbm, t_hbm, m_hbm, wt_hbm, wm_hbm, out_hbm,
             p_sh, idx_all, t_all, m_all, rows0_v, rows1_v, bf0_v, bf1_v,
             wt_v, wm_v, sem_g0, sem_g1, sem_o0, sem_o1):
    sid = lax.axis_index("s")
    wid = sid * NC + lax.axis_index("c")
    base0 = wid * ROWS_PER_W
    rows = (rows0_v, rows1_v)
    bfs = (bf0_v, bf1_v)
    sem_g = (sem_g0, sem_g1)
    sem_o = (sem_o0, sem_o1)

    @pl.when(sid == 0)
    def _():
        pltpu.sync_copy(p_hbm, p_sh)

    pltpu.sync_copy(wt_hbm, wt_v)
    pltpu.sync_copy(wm_hbm, wm_v)
    wts = [wt_v[pl.ds(j * L, L)] for j in range(D // L)]
    wms = [wm_v[pl.ds(j * L, L)] for j in range(D // L)]

    pltpu.sync_copy(atom_hbm.at[pl.ds(base0, ROWS_PER_W)], idx_all)
    pltpu.sync_copy(t_hbm.at[pl.ds(base0, ROWS_PER_W)], t_all)
    pltpu.sync_copy(m_hbm.at[pl.ds(base0, ROWS_PER_W)], m_all)
    plsc.subcore_barrier()

    def gather(ci, b):
        for s in range(N_SUB):
            pltpu.async_copy(
                p_sh.at[idx_all.at[pl.ds(ci * CHUNK + s * SUB, SUB)]],
                bfs[b].at[pl.ds(s * SUB, SUB)],
                sem_g[b])

    def store_out(ci, b):
        pltpu.async_copy(
            rows[b], out_hbm.at[pl.ds(base0 + ci * CHUNK, CHUNK)], sem_o[b])

    def wait_gather(b):
        for s in range(N_SUB):
            pltpu.make_async_copy(
                p_sh.at[idx_all.at[pl.ds(0, SUB)]],
                bfs[b].at[pl.ds(s * SUB, SUB)], sem_g[b]).wait()

    def wait_store(b):
        pltpu.make_async_copy(
            rows[b], out_hbm.at[pl.ds(0, CHUNK)], sem_o[b]).wait()

    def compute(ci, b):
        def group_body(g, c2):
            off = ci * CHUNK + g * L
            tv = t_all[pl.ds(off, L)]
            mv = m_all[pl.ds(off, L)]
            for r in range(L):
                t16 = _lane_bcast(tv, r)
                m16 = _lane_bcast(mv, r)
                row = g * L + r
                for j in range(D // L):
                    v = bfs[b][row, pl.ds(j * L, L)]
                    rows[b][row, pl.ds(j * L, L)] = (
                        v + t16 * wts[j] + m16 * wms[j])
            return c2

        lax.fori_loop(0, CHUNK // L, group_body, 0)

    gather(0, 0)

    @pl.loop(0, N_CHUNKS, step=2)
    def chunk_pair(i0):
        gather(i0 + 1, 1)
        wait_gather(0)

        @pl.when(i0 >= 2)
        def _():
            wait_store(0)
        compute(i0, 0)
        store_out(i0, 0)

        @pl.when(i0 + 2 < N_CHUNKS)
        def _():
            gather(i0 + 2, 0)
        wait_gather(1)

        @pl.when(i0 >= 1)
        def _():
            wait_store(1)
        compute(i0 + 1, 1)
        store_out(i0 + 1, 1)

    wait_store(0)
    wait_store(1)


_sc_lookup = functools.partial(
    pl.kernel,
    out_type=jax.ShapeDtypeStruct((N_ROWS, D), jnp.float32),
    mesh=plsc.VectorSubcoreMesh(core_axis_name="c", subcore_axis_name="s"),
    scratch_types=[
        pltpu.VMEM_SHARED((N_EMB, D), jnp.float32),
        pltpu.VMEM((ROWS_PER_W,), jnp.int32),
        pltpu.VMEM((ROWS_PER_W,), jnp.float32),
        pltpu.VMEM((ROWS_PER_W,), jnp.float32),
        pltpu.VMEM((CHUNK, D), jnp.float32),
        pltpu.VMEM((CHUNK, D), jnp.float32),
        pltpu.VMEM((CHUNK, D), jnp.float32),
        pltpu.VMEM((CHUNK, D), jnp.float32),
        pltpu.VMEM((D,), jnp.float32),
        pltpu.VMEM((D,), jnp.float32),
        pltpu.SemaphoreType.DMA,
        pltpu.SemaphoreType.DMA,
        pltpu.SemaphoreType.DMA,
        pltpu.SemaphoreType.DMA,
    ],
)(_sc_body)


def kernel(atom, time, mag, emb_table, W, b):
    p = _project(emb_table, W, b.reshape(1, D))
    out = _sc_lookup(p, atom.reshape(-1), time.reshape(-1),
                     mag.reshape(-1), W[:, EMB_DIM], W[:, EMB_DIM + 1])
    return out
